# baseline (speedup 1.0000x reference)
import jax
import jax.numpy as jnp
from jax import lax
from jax.experimental import pallas as pl
from jax.experimental.pallas import tpu as pltpu
from jax.experimental.pallas import tpu_sc as plsc

_NC, _NS = 2, 16
_NW = _NC * _NS
_L = 16


def _lane_gather(x, idx):
    return lax.gather(
        x, idx[:, None],
        lax.GatherDimensionNumbers(
            offset_dims=(), collapsed_slice_dims=(0,), start_index_map=(0,)),
        slice_sizes=(1,),
        mode=lax.GatherScatterMode.PROMISE_IN_BOUNDS)


def kernel(q, k, v, kv_cache, slope_rate, slot_idx):
    B, H, _, D = q.shape
    assert B % _NW == 0 and D % _L == 0 and H == _L
    BPW = B // _NW
    HH = H // 2
    EG = D // _L
    DG = D // _L

    q3 = q.reshape(B, H, D)
    k3 = k.reshape(B, H, D)
    v3 = v.reshape(B, H, D)

    mesh = plsc.VectorSubcoreMesh(
        core_axis_name="c", subcore_axis_name="s",
        num_cores=_NC, num_subcores=_NS)

    def body(q_hbm, k_hbm, v_hbm, kv_hbm, slope_hbm, slot_hbm, out_hbm,
             slots_v, decay_v, q_v, k_v, v_v, kvb0, kvb1, out_v,
             sem0, sem1):
        wid = lax.axis_index("s") * _NC + lax.axis_index("c")
        pltpu.sync_copy(slot_hbm, slots_v.at[pl.ds(0, B)])
        pltpu.sync_copy(slope_hbm, decay_v.at[pl.ds(0, H)])
        decay_v[pl.ds(0, H)] = jnp.exp(-decay_v[pl.ds(0, H)])

        for bi in range(BPW):
            b = wid * BPW + bi
            pltpu.sync_copy(q_hbm.at[b], q_v)
            pltpu.sync_copy(k_hbm.at[b], k_v)
            pltpu.sync_copy(v_hbm.at[b], v_v)
            slot = slots_v[pl.ds(b, _L)][0]
            vmask = jnp.where(slot >= 0, 1.0, 0.0)
            sslot = jnp.maximum(slot, 0)
            cp0 = pltpu.make_async_copy(
                kv_hbm.at[sslot, pl.ds(0, HH)], kvb0, sem0)
            cp1 = pltpu.make_async_copy(
                kv_hbm.at[sslot, pl.ds(HH, HH)], kvb1, sem1)
            cp0.start()
            cp1.start()
            for half in range(2):
                (cp0 if half == 0 else cp1).wait()
                buf = kvb0 if half == 0 else kvb1

                def hbody(hh, _, _buf=buf, _half=half):
                    h = _half * HH + hh
                    qk_acc = jnp.zeros((_L,), jnp.float32)
                    for c in range(EG):
                        qk_acc = qk_acc + (q_v[h, pl.ds(c * _L, _L)]
                                           * k_v[h, pl.ds(c * _L, _L)])
                    lane = lax.iota(jnp.int32, _L)
                    for stride in (1, 2, 4, 8):
                        qk_acc = qk_acc + _lane_gather(qk_acc, lane ^ stride)
                    qk = qk_acc
                    dec = _lane_gather(decay_v[pl.ds(h, _L)],
                                       jnp.zeros((_L,), jnp.int32))

                    def dbody(dg, accs):
                        qvec = q_v[h, pl.ds(dg * _L, _L)]
                        for j in range(_L):
                            qd = qvec[j]
                            d = dg * _L + j
                            accs = tuple(
                                accs[eg] + qd * _buf[hh, d, pl.ds(eg * _L, _L)]
                                for eg in range(EG))
                        return accs

                    accs = lax.fori_loop(
                        0, DG, dbody,
                        tuple(jnp.zeros((_L,), jnp.float32)
                              for _ in range(EG)))
                    for eg in range(EG):
                        o = dec * accs[eg] + qk * v_v[h, pl.ds(eg * _L, _L)]
                        out_v[h, pl.ds(eg * _L, _L)] = o * vmask
                    return 0

                lax.fori_loop(0, HH, hbody, 0)
            pltpu.sync_copy(out_v, out_hbm.at[b])

    kfn = pl.kernel(
        body,
        out_type=jax.ShapeDtypeStruct((B, H, D), jnp.float32),
        mesh=mesh,
        compiler_params=pltpu.CompilerParams(use_tc_tiling_on_sc=False),
        scratch_types=[
            pltpu.VMEM((B + _L,), jnp.int32),
            pltpu.VMEM((H + _L,), jnp.float32),
            pltpu.VMEM((H, D), jnp.float32),
            pltpu.VMEM((H, D), jnp.float32),
            pltpu.VMEM((H, D), jnp.float32),
            pltpu.VMEM((HH, D, D), jnp.float32),
            pltpu.VMEM((HH, D, D), jnp.float32),
            pltpu.VMEM((H, D), jnp.float32),
            pltpu.SemaphoreType.DMA,
            pltpu.SemaphoreType.DMA,
        ],
    )
    out = kfn(q3, k3, v3, kv_cache, slope_rate, slot_idx)
    return out[:, :, None, :]
